# baseline (speedup 1.0000x reference)
import jax
import jax.numpy as jnp
from jax import lax
from jax.experimental import pallas as pl
from jax.experimental.pallas import tpu as pltpu
from jax.experimental.pallas import tpu_sc as plsc

N = 10000
NPAD = 10240
E = 160000
EPAD = 163840
NW = 32
EPW = EPAD // NW
CH = 128
NCH = EPW // CH
RPS = NPAD // 16
RB = 1024
RBN = 1000
F_IN, F_HID, F_OUT = 300, 128, 64


def _make_sc_agg(F, with_deg):
    mesh = plsc.VectorSubcoreMesh(core_axis_name="c", subcore_axis_name="s")
    out_type = [jax.ShapeDtypeStruct((2, NPAD, F), jnp.float32)]
    scratch = [
        pltpu.VMEM_SHARED((NPAD, F), jnp.float32),
        pltpu.VMEM((EPW,), jnp.int32),
        pltpu.VMEM((NCH, CH), jnp.int32),
        pltpu.VMEM((CH, F), jnp.float32),
        pltpu.VMEM((CH, F), jnp.float32),
        pltpu.SemaphoreType.DMA,
        pltpu.SemaphoreType.DMA,
        pltpu.SemaphoreType.DMA,
        pltpu.SemaphoreType.DMA,
    ]
    if with_deg:
        out_type.append(jax.ShapeDtypeStruct((2, NPAD), jnp.float32))
        scratch += [
            pltpu.VMEM_SHARED((NPAD,), jnp.float32),
            pltpu.VMEM((CH,), jnp.float32),
        ]

    def body(*refs):
        if with_deg:
            (hn, srcs, dsts, out_p, out_deg,
             acc_s, src_v, dst_v, rows0, rows1,
             semg0, semg1, sems0, sems1, deg_s, ones_v) = refs
        else:
            (hn, srcs, dsts, out_p,
             acc_s, src_v, dst_v, rows0, rows1,
             semg0, semg1, sems0, sems1) = refs
        bufs = (rows0, rows1)
        gsems = (semg0, semg1)
        ssems = (sems0, sems1)
        NB = 2
        c = lax.axis_index("c")
        s = lax.axis_index("s")
        wid = s * 2 + c
        base = s * RPS

        pltpu.sync_copy(srcs.at[wid], src_v)
        pltpu.sync_copy(dsts.at[wid], dst_v)

        def zrow(j, carry):
            for k in range(F // 16):
                rows0[j, pl.ds(k * 16, 16)] = jnp.zeros((16,), jnp.float32)
            return carry
        lax.fori_loop(0, CH, zrow, 0)
        for m in range(RPS // CH):
            pltpu.sync_copy(rows0, acc_s.at[pl.ds(base + m * CH, CH)])
        if with_deg:
            for m in range(RPS // CH):
                pltpu.sync_copy(rows0.at[0], deg_s.at[pl.ds(base + m * CH, CH)])
            for i in range(CH // 16):
                ones_v[pl.ds(i * 16, 16)] = jnp.ones((16,), jnp.float32)
        plsc.subcore_barrier()

        def gather(j, b):
            pltpu.async_copy(hn.at[src_v.at[pl.ds(j * CH, CH)]],
                             bufs[b], gsems[b])

        def wait_gather(j, b):
            pltpu.make_async_copy(hn.at[src_v.at[pl.ds(j * CH, CH)]],
                                  bufs[b], gsems[b]).wait()

        def scatter(j, b):
            pltpu.async_copy(bufs[b], acc_s.at[dst_v.at[j]], ssems[b],
                             add=True)

        def wait_scatter(j, b):
            pltpu.make_async_copy(bufs[b], acc_s.at[dst_v.at[j]],
                                  ssems[b]).wait()

        def scatter_deg(j):
            pltpu.sync_copy(ones_v, deg_s.at[dst_v.at[j]], add=True)

        for b in range(NB):
            gather(b, b)

        def step(i, carry):
            j = i * NB
            for b in range(NB):
                wait_gather(j + b, b)
                scatter(j + b, b)
            if with_deg:
                for b in range(NB):
                    scatter_deg(j + b)
            for b in range(NB):
                wait_scatter(j + b, b)
                gather(j + b + NB, b)
            return carry

        lax.fori_loop(0, NCH // NB - 1, step, 0)
        j = NCH - NB
        for b in range(NB):
            wait_gather(j + b, b)
            scatter(j + b, b)
        if with_deg:
            for b in range(NB):
                scatter_deg(j + b)
        for b in range(NB):
            wait_scatter(j + b, b)
        plsc.subcore_barrier()

        pltpu.sync_copy(acc_s.at[pl.ds(base, RPS)], out_p.at[c, pl.ds(base, RPS)])
        if with_deg:
            pltpu.sync_copy(deg_s.at[pl.ds(base, RPS)],
                            out_deg.at[c, pl.ds(base, RPS)])

    return pl.kernel(body, out_type=out_type, scratch_types=scratch, mesh=mesh)


_agg_hid_deg = _make_sc_agg(F_HID, True)
_agg_hid = _make_sc_agg(F_HID, False)


_T_DN = (((0,), (0,)), ((), ()))


def _mm_in_body(xt_ref, ws_ref, wn_ref, b_ref, hso_ref, hno_ref):
    xt = xt_ref[...]
    hso_ref[...] = lax.dot_general(
        xt, ws_ref[...], _T_DN, preferred_element_type=jnp.float32) + b_ref[...]
    hno_ref[...] = lax.dot_general(
        xt, wn_ref[...], _T_DN, preferred_element_type=jnp.float32)


def _mm_in(xt, ws, wn, b):
    return pl.pallas_call(
        _mm_in_body,
        grid=(NPAD // RB,),
        in_specs=[
            pl.BlockSpec((F_IN, RB), lambda i: (0, i)),
            pl.BlockSpec((F_IN, F_HID), lambda i: (0, 0)),
            pl.BlockSpec((F_IN, F_HID), lambda i: (0, 0)),
            pl.BlockSpec((1, F_HID), lambda i: (0, 0)),
        ],
        out_specs=[pl.BlockSpec((RB, F_HID), lambda i: (i, 0))] * 2,
        out_shape=[jax.ShapeDtypeStruct((NPAD, F_HID), jnp.float32)] * 2,
    )(xt, ws, wn, b.reshape(1, F_HID))


def _combine_mm_body(hs_ref, p_ref, d_ref, ws_ref, wn_ref, b_ref,
                     hso_ref, hno_ref):
    ps = p_ref[0] + p_ref[1]
    d = d_ref[0] + d_ref[1]
    inv = 1.0 / jnp.maximum(d, 1.0)
    h = jnp.maximum(hs_ref[...] + ps * inv[:, None], 0.0)
    hso_ref[...] = jnp.dot(h, ws_ref[...],
                           preferred_element_type=jnp.float32) + b_ref[...]
    hno_ref[...] = jnp.dot(h, wn_ref[...], preferred_element_type=jnp.float32)


def _combine_mm(hs, p, degp, ws, wn, b, fs, fn):
    return pl.pallas_call(
        _combine_mm_body,
        grid=(NPAD // RB,),
        in_specs=[
            pl.BlockSpec((RB, F_HID), lambda i: (i, 0)),
            pl.BlockSpec((2, RB, F_HID), lambda i: (0, i, 0)),
            pl.BlockSpec((2, RB), lambda i: (0, i)),
            pl.BlockSpec((F_HID, fs), lambda i: (0, 0)),
            pl.BlockSpec((F_HID, fn), lambda i: (0, 0)),
            pl.BlockSpec((1, fs), lambda i: (0, 0)),
        ],
        out_specs=[pl.BlockSpec((RB, fs), lambda i: (i, 0)),
                   pl.BlockSpec((RB, fn), lambda i: (i, 0))],
        out_shape=[jax.ShapeDtypeStruct((NPAD, fs), jnp.float32),
                   jax.ShapeDtypeStruct((NPAD, fn), jnp.float32)],
    )(hs, p, degp, ws, wn, b.reshape(1, fs))


def _final_body(hs_ref, p_ref, d_ref, o_ref):
    ps = p_ref[0, :, :F_OUT] + p_ref[1, :, :F_OUT]
    d = d_ref[0] + d_ref[1]
    o_ref[...] = hs_ref[...] + ps * (1.0 / jnp.maximum(d, 1.0))[:, None]


def _final(hs, p, degp):
    return pl.pallas_call(
        _final_body,
        grid=(NPAD // RB,),
        in_specs=[
            pl.BlockSpec((RB, F_OUT), lambda i: (i, 0)),
            pl.BlockSpec((2, RB, F_HID), lambda i: (0, i, 0)),
            pl.BlockSpec((2, RB), lambda i: (0, i)),
        ],
        out_specs=pl.BlockSpec((RB, F_OUT), lambda i: (i, 0)),
        out_shape=jax.ShapeDtypeStruct((N, F_OUT), jnp.float32),
    )(hs, p, degp)


def kernel(x, edge_index, W_self_0, W_neigh_0, b_0, W_self_1, W_neigh_1, b_1,
           W_self_2, W_neigh_2, b_2):
    xt = x.reshape(-1, F_IN).T
    src = edge_index[0].astype(jnp.int32)
    dst = edge_index[1].astype(jnp.int32)
    fill = (jnp.arange(EPAD - E, dtype=jnp.int32) % (NPAD - N)) + N
    srcs = jnp.concatenate([src, fill]).reshape(NW, EPW)
    dsts = jnp.concatenate([dst, fill]).reshape(NW, NCH, CH)
    wn2 = jnp.pad(W_neigh_2, ((0, 0), (0, F_HID - F_OUT)))

    hs0, hn0 = _mm_in(xt, W_self_0, W_neigh_0, b_0)
    p0, degp = _agg_hid_deg(hn0, srcs, dsts)
    hs1, hn1 = _combine_mm(hs0, p0, degp, W_self_1, W_neigh_1, b_1, F_HID, F_HID)
    (p1,) = _agg_hid(hn1, srcs, dsts)
    hs2, hn2 = _combine_mm(hs1, p1, degp, W_self_2, wn2, b_2, F_OUT, F_HID)
    (p2,) = _agg_hid(hn2, srcs, dsts)
    out = _final(hs2, p2, degp)
    return out


# speedup vs baseline: 1.2640x; 1.2640x over previous
import jax
import jax.numpy as jnp
from jax import lax
from jax.experimental import pallas as pl
from jax.experimental.pallas import tpu as pltpu
from jax.experimental.pallas import tpu_sc as plsc

N = 10000
NPAD = 10240
E = 160000
EPAD = 163840
NW = 32
EPW = EPAD // NW
CH = 128
NCH = EPW // CH
RPS = NPAD // 16
RB = 2048
F_IN, F_HID, F_OUT = 300, 128, 64


def _make_sc_agg(F, with_deg):
    mesh = plsc.VectorSubcoreMesh(core_axis_name="c", subcore_axis_name="s")
    out_type = [jax.ShapeDtypeStruct((2, NPAD, F), jnp.float32)]
    scratch = [
        pltpu.VMEM_SHARED((NPAD, F), jnp.float32),
        pltpu.VMEM((EPW,), jnp.int32),
        pltpu.VMEM((NCH, CH), jnp.int32),
        pltpu.VMEM((CH, F), jnp.float32),
        pltpu.VMEM((CH, F), jnp.float32),
        pltpu.SemaphoreType.DMA,
        pltpu.SemaphoreType.DMA,
        pltpu.SemaphoreType.DMA,
        pltpu.SemaphoreType.DMA,
    ]
    if with_deg:
        out_type.append(jax.ShapeDtypeStruct((2, NPAD), jnp.float32))
        scratch += [
            pltpu.VMEM_SHARED((NPAD,), jnp.float32),
            pltpu.VMEM((CH,), jnp.float32),
        ]

    def body(*refs):
        if with_deg:
            (hn, srcs, dsts, out_p, out_deg,
             acc_s, src_v, dst_v, rows0, rows1,
             semg0, semg1, sems0, sems1, deg_s, ones_v) = refs
        else:
            (hn, srcs, dsts, out_p,
             acc_s, src_v, dst_v, rows0, rows1,
             semg0, semg1, sems0, sems1) = refs
        bufs = (rows0, rows1)
        gsems = (semg0, semg1)
        ssems = (sems0, sems1)
        NB = 2
        c = lax.axis_index("c")
        s = lax.axis_index("s")
        wid = s * 2 + c
        base = s * RPS

        pltpu.sync_copy(srcs.at[wid], src_v)
        pltpu.sync_copy(dsts.at[wid], dst_v)

        def zrow(j, carry):
            for k in range(F // 16):
                rows0[j, pl.ds(k * 16, 16)] = jnp.zeros((16,), jnp.float32)
            return carry
        lax.fori_loop(0, CH, zrow, 0)
        for m in range(RPS // CH):
            pltpu.sync_copy(rows0, acc_s.at[pl.ds(base + m * CH, CH)])
        if with_deg:
            for m in range(RPS // CH):
                pltpu.sync_copy(rows0.at[0], deg_s.at[pl.ds(base + m * CH, CH)])
            for i in range(CH // 16):
                ones_v[pl.ds(i * 16, 16)] = jnp.ones((16,), jnp.float32)
        plsc.subcore_barrier()

        def gather(j, b):
            pltpu.async_copy(hn.at[src_v.at[pl.ds(j * CH, CH)]],
                             bufs[b], gsems[b])

        def wait_gather(j, b):
            pltpu.make_async_copy(hn.at[src_v.at[pl.ds(j * CH, CH)]],
                                  bufs[b], gsems[b]).wait()

        def scatter(j, b):
            pltpu.sync_copy(bufs[b], acc_s.at[dst_v.at[j]], add=True)
            if with_deg:
                pltpu.sync_copy(ones_v, deg_s.at[dst_v.at[j]], add=True)

        for b in range(NB):
            gather(b, b)

        def step(i, carry):
            j = i * NB
            for b in range(NB):
                wait_gather(j + b, b)
                scatter(j + b, b)
                gather(j + b + NB, b)
            return carry

        lax.fori_loop(0, NCH // NB - 1, step, 0)
        j = NCH - NB
        for b in range(NB):
            wait_gather(j + b, b)
            scatter(j + b, b)
        plsc.subcore_barrier()

        pltpu.sync_copy(acc_s.at[pl.ds(base, RPS)], out_p.at[c, pl.ds(base, RPS)])
        if with_deg:
            pltpu.sync_copy(deg_s.at[pl.ds(base, RPS)],
                            out_deg.at[c, pl.ds(base, RPS)])

    return pl.kernel(body, out_type=out_type, scratch_types=scratch, mesh=mesh)


_agg_hid_deg = _make_sc_agg(F_HID, True)
_agg_hid = _make_sc_agg(F_HID, False)


_T_DN = (((0,), (0,)), ((), ()))


def _mm_in_body(xt_ref, ws_ref, wn_ref, b_ref, hso_ref, hno_ref):
    xt = xt_ref[...]
    hso_ref[...] = lax.dot_general(
        xt, ws_ref[...], _T_DN, preferred_element_type=jnp.float32) + b_ref[...]
    hno_ref[...] = lax.dot_general(
        xt, wn_ref[...], _T_DN, preferred_element_type=jnp.float32)


def _mm_in(xt, ws, wn, b):
    return pl.pallas_call(
        _mm_in_body,
        grid=(NPAD // RB,),
        in_specs=[
            pl.BlockSpec((F_IN, RB), lambda i: (0, i)),
            pl.BlockSpec((F_IN, F_HID), lambda i: (0, 0)),
            pl.BlockSpec((F_IN, F_HID), lambda i: (0, 0)),
            pl.BlockSpec((1, F_HID), lambda i: (0, 0)),
        ],
        out_specs=[pl.BlockSpec((RB, F_HID), lambda i: (i, 0))] * 2,
        out_shape=[jax.ShapeDtypeStruct((NPAD, F_HID), jnp.float32)] * 2,
    )(xt, ws, wn, b.reshape(1, F_HID))


def _combine_mm_body(hs_ref, p_ref, d_ref, ws_ref, wn_ref, b_ref,
                     hso_ref, hno_ref):
    ps = p_ref[0] + p_ref[1]
    d = d_ref[0] + d_ref[1]
    inv = 1.0 / jnp.maximum(d, 1.0)
    h = jnp.maximum(hs_ref[...] + ps * inv[:, None], 0.0)
    hso_ref[...] = jnp.dot(h, ws_ref[...],
                           preferred_element_type=jnp.float32) + b_ref[...]
    hno_ref[...] = jnp.dot(h, wn_ref[...], preferred_element_type=jnp.float32)


def _combine_mm(hs, p, degp, ws, wn, b, fs, fn):
    return pl.pallas_call(
        _combine_mm_body,
        grid=(NPAD // RB,),
        in_specs=[
            pl.BlockSpec((RB, F_HID), lambda i: (i, 0)),
            pl.BlockSpec((2, RB, F_HID), lambda i: (0, i, 0)),
            pl.BlockSpec((2, RB), lambda i: (0, i)),
            pl.BlockSpec((F_HID, fs), lambda i: (0, 0)),
            pl.BlockSpec((F_HID, fn), lambda i: (0, 0)),
            pl.BlockSpec((1, fs), lambda i: (0, 0)),
        ],
        out_specs=[pl.BlockSpec((RB, fs), lambda i: (i, 0)),
                   pl.BlockSpec((RB, fn), lambda i: (i, 0))],
        out_shape=[jax.ShapeDtypeStruct((NPAD, fs), jnp.float32),
                   jax.ShapeDtypeStruct((NPAD, fn), jnp.float32)],
    )(hs, p, degp, ws, wn, b.reshape(1, fs))


def _final_body(hs_ref, p_ref, d_ref, o_ref):
    ps = p_ref[0, :, :F_OUT] + p_ref[1, :, :F_OUT]
    d = d_ref[0] + d_ref[1]
    o_ref[...] = hs_ref[...] + ps * (1.0 / jnp.maximum(d, 1.0))[:, None]


def _final(hs, p, degp):
    return pl.pallas_call(
        _final_body,
        grid=(NPAD // RB,),
        in_specs=[
            pl.BlockSpec((RB, F_OUT), lambda i: (i, 0)),
            pl.BlockSpec((2, RB, F_HID), lambda i: (0, i, 0)),
            pl.BlockSpec((2, RB), lambda i: (0, i)),
        ],
        out_specs=pl.BlockSpec((RB, F_OUT), lambda i: (i, 0)),
        out_shape=jax.ShapeDtypeStruct((N, F_OUT), jnp.float32),
    )(hs, p, degp)


def kernel(x, edge_index, W_self_0, W_neigh_0, b_0, W_self_1, W_neigh_1, b_1,
           W_self_2, W_neigh_2, b_2):
    xt = x.reshape(-1, F_IN).T
    src = edge_index[0].astype(jnp.int32)
    dst = edge_index[1].astype(jnp.int32)
    fill = (jnp.arange(EPAD - E, dtype=jnp.int32) % (NPAD - N)) + N
    srcs = jnp.concatenate([src, fill]).reshape(NW, EPW)
    dsts = jnp.concatenate([dst, fill]).reshape(NW, NCH, CH)
    wn2 = jnp.pad(W_neigh_2, ((0, 0), (0, F_HID - F_OUT)))

    hs0, hn0 = _mm_in(xt, W_self_0, W_neigh_0, b_0)
    p0, degp = _agg_hid_deg(hn0, srcs, dsts)
    hs1, hn1 = _combine_mm(hs0, p0, degp, W_self_1, W_neigh_1, b_1, F_HID, F_HID)
    (p1,) = _agg_hid(hn1, srcs, dsts)
    hs2, hn2 = _combine_mm(hs1, p1, degp, W_self_2, wn2, b_2, F_OUT, F_HID)
    (p2,) = _agg_hid(hn2, srcs, dsts)
    out = _final(hs2, p2, degp)
    return out


# speedup vs baseline: 1.2996x; 1.0282x over previous
import jax
import jax.numpy as jnp
from jax import lax
from jax.experimental import pallas as pl
from jax.experimental.pallas import tpu as pltpu
from jax.experimental.pallas import tpu_sc as plsc

N = 10000
NPAD = 10240
E = 160000
EPAD = 163840
NW = 32
EPW = EPAD // NW
CH = 128
NCH = EPW // CH
RPS = NPAD // 16
RB = 2048
F_IN, F_HID, F_OUT = 300, 128, 64


def _make_sc_agg(F, with_deg):
    mesh = plsc.VectorSubcoreMesh(core_axis_name="c", subcore_axis_name="s")
    out_type = [jax.ShapeDtypeStruct((2, NPAD, F), jnp.float32)]
    scratch = [
        pltpu.VMEM_SHARED((NPAD, F), jnp.float32),
        pltpu.VMEM((NCH, CH), jnp.int32),
        pltpu.VMEM((NCH, CH), jnp.int32),
        pltpu.VMEM((CH, F), jnp.float32),
        pltpu.VMEM((CH, F), jnp.float32),
        pltpu.SemaphoreType.DMA,
        pltpu.SemaphoreType.DMA,
        pltpu.SemaphoreType.DMA,
        pltpu.SemaphoreType.DMA,
    ]
    if with_deg:
        out_type.append(jax.ShapeDtypeStruct((2, NPAD), jnp.float32))
        scratch += [
            pltpu.VMEM_SHARED((NPAD,), jnp.float32),
            pltpu.VMEM((CH,), jnp.float32),
        ]

    def body(*refs):
        if with_deg:
            (hn, edges, out_p, out_deg,
             acc_s, src_v, dst_v, rows0, rows1,
             semg0, semg1, sems0, sems1, deg_s, ones_v) = refs
        else:
            (hn, edges, out_p,
             acc_s, src_v, dst_v, rows0, rows1,
             semg0, semg1, sems0, sems1) = refs
        bufs = (rows0, rows1)
        gsems = (semg0, semg1)
        ssems = (sems0, sems1)
        NB = 2
        c = lax.axis_index("c")
        s = lax.axis_index("s")
        wid = s * 2 + c
        base = s * RPS

        pltpu.sync_copy(edges.at[0, wid], src_v)
        pltpu.sync_copy(edges.at[1, wid], dst_v)

        def zrow(j, carry):
            for k in range(F // 16):
                rows0[j, pl.ds(k * 16, 16)] = jnp.zeros((16,), jnp.float32)
            return carry
        lax.fori_loop(0, CH, zrow, 0)
        for m in range(RPS // CH):
            pltpu.sync_copy(rows0, acc_s.at[pl.ds(base + m * CH, CH)])
        if with_deg:
            for m in range(RPS // CH):
                pltpu.sync_copy(rows0.at[0], deg_s.at[pl.ds(base + m * CH, CH)])
            for i in range(CH // 16):
                ones_v[pl.ds(i * 16, 16)] = jnp.ones((16,), jnp.float32)
        plsc.subcore_barrier()

        def gather(j, b):
            pltpu.async_copy(hn.at[src_v.at[j]], bufs[b], gsems[b])

        def wait_gather(j, b):
            pltpu.make_async_copy(hn.at[src_v.at[j]], bufs[b], gsems[b]).wait()

        def scatter(j, b):
            pltpu.sync_copy(bufs[b], acc_s.at[dst_v.at[j]], add=True)
            if with_deg:
                pltpu.sync_copy(ones_v, deg_s.at[dst_v.at[j]], add=True)

        for b in range(NB):
            gather(b, b)

        def step(i, carry):
            j = i * NB
            for b in range(NB):
                wait_gather(j + b, b)
                scatter(j + b, b)
                gather(j + b + NB, b)
            return carry

        lax.fori_loop(0, NCH // NB - 1, step, 0)
        j = NCH - NB
        for b in range(NB):
            wait_gather(j + b, b)
            scatter(j + b, b)
        plsc.subcore_barrier()

        pltpu.sync_copy(acc_s.at[pl.ds(base, RPS)], out_p.at[c, pl.ds(base, RPS)])
        if with_deg:
            pltpu.sync_copy(deg_s.at[pl.ds(base, RPS)],
                            out_deg.at[c, pl.ds(base, RPS)])

    return pl.kernel(body, out_type=out_type, scratch_types=scratch, mesh=mesh)


_agg_hid_deg = _make_sc_agg(F_HID, True)
_agg_hid = _make_sc_agg(F_HID, False)


_T_DN = (((0,), (0,)), ((), ()))


def _mm_in_body(xt_ref, ws_ref, wn_ref, b_ref, hso_ref, hno_ref):
    xt = xt_ref[...]
    hso_ref[...] = lax.dot_general(
        xt, ws_ref[...], _T_DN, preferred_element_type=jnp.float32) + b_ref[...]
    hno_ref[...] = lax.dot_general(
        xt, wn_ref[...], _T_DN, preferred_element_type=jnp.float32)


def _mm_in(xt, ws, wn, b):
    return pl.pallas_call(
        _mm_in_body,
        grid=(NPAD // RB,),
        in_specs=[
            pl.BlockSpec((F_IN, RB), lambda i: (0, i)),
            pl.BlockSpec((F_IN, F_HID), lambda i: (0, 0)),
            pl.BlockSpec((F_IN, F_HID), lambda i: (0, 0)),
            pl.BlockSpec((1, F_HID), lambda i: (0, 0)),
        ],
        out_specs=[pl.BlockSpec((RB, F_HID), lambda i: (i, 0))] * 2,
        out_shape=[jax.ShapeDtypeStruct((NPAD, F_HID), jnp.float32)] * 2,
    )(xt, ws, wn, b.reshape(1, F_HID))


def _combine_mm_body(hs_ref, p_ref, d_ref, ws_ref, wn_ref, b_ref,
                     hso_ref, hno_ref):
    ps = p_ref[0] + p_ref[1]
    d = d_ref[0] + d_ref[1]
    inv = 1.0 / jnp.maximum(d, 1.0)
    h = jnp.maximum(hs_ref[...] + ps * inv[:, None], 0.0)
    hso_ref[...] = jnp.dot(h, ws_ref[...],
                           preferred_element_type=jnp.float32) + b_ref[...]
    hno_ref[...] = jnp.dot(h, wn_ref[...], preferred_element_type=jnp.float32)


def _combine_mm(hs, p, degp, ws, wn, b, fs, fn):
    return pl.pallas_call(
        _combine_mm_body,
        grid=(NPAD // RB,),
        in_specs=[
            pl.BlockSpec((RB, F_HID), lambda i: (i, 0)),
            pl.BlockSpec((2, RB, F_HID), lambda i: (0, i, 0)),
            pl.BlockSpec((2, RB), lambda i: (0, i)),
            pl.BlockSpec((F_HID, fs), lambda i: (0, 0)),
            pl.BlockSpec((F_HID, fn), lambda i: (0, 0)),
            pl.BlockSpec((1, fs), lambda i: (0, 0)),
        ],
        out_specs=[pl.BlockSpec((RB, fs), lambda i: (i, 0)),
                   pl.BlockSpec((RB, fn), lambda i: (i, 0))],
        out_shape=[jax.ShapeDtypeStruct((NPAD, fs), jnp.float32),
                   jax.ShapeDtypeStruct((NPAD, fn), jnp.float32)],
    )(hs, p, degp, ws, wn, b.reshape(1, fs))


def _final_body(hs_ref, p_ref, d_ref, o_ref):
    ps = p_ref[0, :, :F_OUT] + p_ref[1, :, :F_OUT]
    d = d_ref[0] + d_ref[1]
    o_ref[...] = hs_ref[...] + ps * (1.0 / jnp.maximum(d, 1.0))[:, None]


def _final(hs, p, degp):
    return pl.pallas_call(
        _final_body,
        grid=(NPAD // RB,),
        in_specs=[
            pl.BlockSpec((RB, F_OUT), lambda i: (i, 0)),
            pl.BlockSpec((2, RB, F_HID), lambda i: (0, i, 0)),
            pl.BlockSpec((2, RB), lambda i: (0, i)),
        ],
        out_specs=pl.BlockSpec((RB, F_OUT), lambda i: (i, 0)),
        out_shape=jax.ShapeDtypeStruct((N, F_OUT), jnp.float32),
    )(hs, p, degp)


def kernel(x, edge_index, W_self_0, W_neigh_0, b_0, W_self_1, W_neigh_1, b_1,
           W_self_2, W_neigh_2, b_2):
    xt = x.reshape(-1, F_IN).T
    fill = (jnp.arange(EPAD - E, dtype=jnp.int32) % (NPAD - N)) + N
    fill2 = jnp.broadcast_to(fill, (2, EPAD - E))
    edges = jnp.concatenate([edge_index.astype(jnp.int32), fill2],
                            axis=1).reshape(2, NW, NCH, CH)
    wn2 = jnp.pad(W_neigh_2, ((0, 0), (0, F_HID - F_OUT)))

    hs0, hn0 = _mm_in(xt, W_self_0, W_neigh_0, b_0)
    p0, degp = _agg_hid_deg(hn0, edges)
    hs1, hn1 = _combine_mm(hs0, p0, degp, W_self_1, W_neigh_1, b_1, F_HID, F_HID)
    (p1,) = _agg_hid(hn1, edges)
    hs2, hn2 = _combine_mm(hs1, p1, degp, W_self_2, wn2, b_2, F_OUT, F_HID)
    (p2,) = _agg_hid(hn2, edges)
    out = _final(hs2, p2, degp)
    return out


# speedup vs baseline: 1.3044x; 1.0037x over previous
import jax
import jax.numpy as jnp
from jax import lax
from jax.experimental import pallas as pl
from jax.experimental.pallas import tpu as pltpu
from jax.experimental.pallas import tpu_sc as plsc

N = 10000
NPAD = 10240
E = 160000
EPAD = 163840
NW = 32
EPW = EPAD // NW
CH = 128
NCH = EPW // CH
RPS = NPAD // 16
RB = 2048
F_IN, F_HID, F_OUT = 300, 128, 64


def _make_sc_agg(F, with_deg):
    mesh = plsc.VectorSubcoreMesh(core_axis_name="c", subcore_axis_name="s")
    out_type = [jax.ShapeDtypeStruct((2, NPAD, F), jnp.float32)]
    scratch = [
        pltpu.VMEM_SHARED((NPAD, F), jnp.float32),
        pltpu.VMEM((NCH, CH), jnp.int32),
        pltpu.VMEM((NCH, CH), jnp.int32),
        pltpu.VMEM((CH, F), jnp.float32),
        pltpu.VMEM((CH, F), jnp.float32),
        pltpu.SemaphoreType.DMA,
        pltpu.SemaphoreType.DMA,
        pltpu.SemaphoreType.DMA,
        pltpu.SemaphoreType.DMA,
    ]
    if with_deg:
        out_type.append(jax.ShapeDtypeStruct((2, NPAD), jnp.float32))
        scratch += [
            pltpu.VMEM_SHARED((NPAD,), jnp.float32),
            pltpu.VMEM((CH,), jnp.float32),
        ]

    def body(*refs):
        if with_deg:
            (hn, edges, out_p, out_deg,
             acc_s, src_v, dst_v, rows0, rows1,
             semg0, semg1, sems0, sems1, deg_s, ones_v) = refs
        else:
            (hn, edges, out_p,
             acc_s, src_v, dst_v, rows0, rows1,
             semg0, semg1, sems0, sems1) = refs
        bufs = (rows0, rows1)
        gsems = (semg0, semg1)
        ssems = (sems0, sems1)
        NB = 2
        c = lax.axis_index("c")
        s = lax.axis_index("s")
        wid = s * 2 + c
        base = s * RPS

        pltpu.sync_copy(edges.at[0, wid], src_v)
        pltpu.sync_copy(edges.at[1, wid], dst_v)

        def zrow(j, carry):
            for k in range(F // 16):
                rows0[j, pl.ds(k * 16, 16)] = jnp.zeros((16,), jnp.float32)
            return carry
        lax.fori_loop(0, CH, zrow, 0)
        for m in range(RPS // CH):
            pltpu.sync_copy(rows0, acc_s.at[pl.ds(base + m * CH, CH)])
        if with_deg:
            for m in range(RPS // CH):
                pltpu.sync_copy(rows0.at[0], deg_s.at[pl.ds(base + m * CH, CH)])
            for i in range(CH // 16):
                ones_v[pl.ds(i * 16, 16)] = jnp.ones((16,), jnp.float32)
        plsc.subcore_barrier()

        def gather(j, b):
            pltpu.async_copy(hn.at[src_v.at[j]], bufs[b], gsems[b])

        def wait_gather(j, b):
            pltpu.make_async_copy(hn.at[src_v.at[j]], bufs[b], gsems[b]).wait()

        def scatter(j, b):
            pltpu.sync_copy(bufs[b], acc_s.at[dst_v.at[j]], add=True)
            if with_deg:
                pltpu.sync_copy(ones_v, deg_s.at[dst_v.at[j]], add=True)

        for b in range(NB):
            gather(b, b)

        def step(i, carry):
            j = i * NB
            for b in range(NB):
                wait_gather(j + b, b)
                scatter(j + b, b)
                gather(j + b + NB, b)
            return carry

        lax.fori_loop(0, NCH // NB - 1, step, 0)
        j = NCH - NB
        for b in range(NB):
            wait_gather(j + b, b)
            scatter(j + b, b)
        plsc.subcore_barrier()

        pltpu.sync_copy(acc_s.at[pl.ds(base, RPS)], out_p.at[c, pl.ds(base, RPS)])
        if with_deg:
            pltpu.sync_copy(deg_s.at[pl.ds(base, RPS)],
                            out_deg.at[c, pl.ds(base, RPS)])

    return pl.kernel(body, out_type=out_type, scratch_types=scratch, mesh=mesh)


_agg_hid_deg = _make_sc_agg(F_HID, True)
_agg_hid = _make_sc_agg(F_HID, False)


_T_DN = (((0,), (0,)), ((), ()))


def _mm_in_one(xt, w, b=None):
    def body(xt_ref, w_ref, *rest):
        if b is None:
            (o_ref,) = rest
            bias = 0.0
        else:
            b_ref, o_ref = rest
            bias = b_ref[...]
        o_ref[...] = lax.dot_general(
            xt_ref[...], w_ref[...], _T_DN,
            preferred_element_type=jnp.float32) + bias

    in_specs = [
        pl.BlockSpec((F_IN, RB), lambda i: (0, i)),
        pl.BlockSpec((F_IN, F_HID), lambda i: (0, 0)),
    ]
    args = [xt, w]
    if b is not None:
        in_specs.append(pl.BlockSpec((1, F_HID), lambda i: (0, 0)))
        args.append(b.reshape(1, F_HID))
    return pl.pallas_call(
        body,
        grid=(NPAD // RB,),
        in_specs=in_specs,
        out_specs=pl.BlockSpec((RB, F_HID), lambda i: (i, 0)),
        out_shape=jax.ShapeDtypeStruct((NPAD, F_HID), jnp.float32),
    )(*args)


def _combine_mm_one(hs, p, degp, w, fo, b=None):
    def body(hs_ref, p_ref, d_ref, w_ref, *rest):
        if b is None:
            (o_ref,) = rest
            bias = 0.0
        else:
            b_ref, o_ref = rest
            bias = b_ref[...]
        ps = p_ref[0] + p_ref[1]
        d = d_ref[0] + d_ref[1]
        inv = 1.0 / jnp.maximum(d, 1.0)
        h = jnp.maximum(hs_ref[...] + ps * inv[:, None], 0.0)
        o_ref[...] = jnp.dot(h, w_ref[...],
                             preferred_element_type=jnp.float32) + bias

    in_specs = [
        pl.BlockSpec((RB, F_HID), lambda i: (i, 0)),
        pl.BlockSpec((2, RB, F_HID), lambda i: (0, i, 0)),
        pl.BlockSpec((2, RB), lambda i: (0, i)),
        pl.BlockSpec((F_HID, fo), lambda i: (0, 0)),
    ]
    args = [hs, p, degp, w]
    if b is not None:
        in_specs.append(pl.BlockSpec((1, fo), lambda i: (0, 0)))
        args.append(b.reshape(1, fo))
    return pl.pallas_call(
        body,
        grid=(NPAD // RB,),
        in_specs=in_specs,
        out_specs=pl.BlockSpec((RB, fo), lambda i: (i, 0)),
        out_shape=jax.ShapeDtypeStruct((NPAD, fo), jnp.float32),
    )(*args)


def _final_body(hs_ref, p_ref, d_ref, o_ref):
    ps = p_ref[0, :, :F_OUT] + p_ref[1, :, :F_OUT]
    d = d_ref[0] + d_ref[1]
    o_ref[...] = hs_ref[...] + ps * (1.0 / jnp.maximum(d, 1.0))[:, None]


def _final(hs, p, degp):
    return pl.pallas_call(
        _final_body,
        grid=(NPAD // RB,),
        in_specs=[
            pl.BlockSpec((RB, F_OUT), lambda i: (i, 0)),
            pl.BlockSpec((2, RB, F_HID), lambda i: (0, i, 0)),
            pl.BlockSpec((2, RB), lambda i: (0, i)),
        ],
        out_specs=pl.BlockSpec((RB, F_OUT), lambda i: (i, 0)),
        out_shape=jax.ShapeDtypeStruct((N, F_OUT), jnp.float32),
    )(hs, p, degp)


def kernel(x, edge_index, W_self_0, W_neigh_0, b_0, W_self_1, W_neigh_1, b_1,
           W_self_2, W_neigh_2, b_2):
    xt = x.reshape(-1, F_IN).T
    fill = (jnp.arange(EPAD - E, dtype=jnp.int32) % (NPAD - N)) + N
    fill2 = jnp.broadcast_to(fill, (2, EPAD - E))
    edges = jnp.concatenate([edge_index.astype(jnp.int32), fill2],
                            axis=1).reshape(2, NW, NCH, CH)
    wn2 = jnp.pad(W_neigh_2, ((0, 0), (0, F_HID - F_OUT)))

    hn0 = _mm_in_one(xt, W_neigh_0)
    p0, degp = _agg_hid_deg(hn0, edges)
    hs0 = _mm_in_one(xt, W_self_0, b_0)
    hn1 = _combine_mm_one(hs0, p0, degp, W_neigh_1, F_HID)
    (p1,) = _agg_hid(hn1, edges)
    hs1 = _combine_mm_one(hs0, p0, degp, W_self_1, F_HID, b_1)
    hn2 = _combine_mm_one(hs1, p1, degp, wn2, F_HID)
    (p2,) = _agg_hid(hn2, edges)
    hs2 = _combine_mm_one(hs1, p1, degp, W_self_2, F_OUT, b_2)
    out = _final(hs2, p2, degp)
    return out


# speedup vs baseline: 1.3550x; 1.0388x over previous
import jax
import jax.numpy as jnp
from jax import lax
from jax.experimental import pallas as pl
from jax.experimental.pallas import tpu as pltpu
from jax.experimental.pallas import tpu_sc as plsc

N = 10000
NPAD = 10240
E = 160000
EPAD = 163840
NW = 32
EPW = EPAD // NW
CH = 128
NCH = EPW // CH
RPS = NPAD // 16
RB = 2560
F_IN, F_HID, F_OUT = 300, 128, 64


def _make_sc_agg(F, with_deg):
    mesh = plsc.VectorSubcoreMesh(core_axis_name="c", subcore_axis_name="s")
    out_type = [jax.ShapeDtypeStruct((2, NPAD, F), jnp.float32)]
    scratch = [
        pltpu.VMEM_SHARED((NPAD, F), jnp.float32),
        pltpu.VMEM((NCH, CH), jnp.int32),
        pltpu.VMEM((NCH, CH), jnp.int32),
        pltpu.VMEM((CH, F), jnp.float32),
        pltpu.VMEM((CH, F), jnp.float32),
        pltpu.SemaphoreType.DMA,
        pltpu.SemaphoreType.DMA,
        pltpu.SemaphoreType.DMA,
        pltpu.SemaphoreType.DMA,
    ]
    if with_deg:
        out_type.append(jax.ShapeDtypeStruct((2, NPAD), jnp.float32))
        scratch += [
            pltpu.VMEM_SHARED((NPAD,), jnp.float32),
            pltpu.VMEM((CH,), jnp.float32),
        ]

    def body(*refs):
        if with_deg:
            (hn, edges, out_p, out_deg,
             acc_s, src_v, dst_v, rows0, rows1,
             semg0, semg1, sems0, sems1, deg_s, ones_v) = refs
        else:
            (hn, edges, out_p,
             acc_s, src_v, dst_v, rows0, rows1,
             semg0, semg1, sems0, sems1) = refs
        bufs = (rows0, rows1)
        gsems = (semg0, semg1)
        ssems = (sems0, sems1)
        NB = 2
        c = lax.axis_index("c")
        s = lax.axis_index("s")
        wid = s * 2 + c
        base = s * RPS

        pltpu.sync_copy(edges.at[0, wid], src_v)
        pltpu.sync_copy(edges.at[1, wid], dst_v)

        def zrow(j, carry):
            for k in range(F // 16):
                rows0[j, pl.ds(k * 16, 16)] = jnp.zeros((16,), jnp.float32)
            return carry
        lax.fori_loop(0, CH, zrow, 0)
        for m in range(RPS // CH):
            pltpu.sync_copy(rows0, acc_s.at[pl.ds(base + m * CH, CH)])
        if with_deg:
            for m in range(RPS // CH):
                pltpu.sync_copy(rows0.at[0], deg_s.at[pl.ds(base + m * CH, CH)])
            for i in range(CH // 16):
                ones_v[pl.ds(i * 16, 16)] = jnp.ones((16,), jnp.float32)
        plsc.subcore_barrier()

        def gather(j, b):
            pltpu.async_copy(hn.at[src_v.at[j]], bufs[b], gsems[b])

        def wait_gather(j, b):
            pltpu.make_async_copy(hn.at[src_v.at[j]], bufs[b], gsems[b]).wait()

        def scatter(j, b):
            pltpu.sync_copy(bufs[b], acc_s.at[dst_v.at[j]], add=True)
            if with_deg:
                pltpu.sync_copy(ones_v, deg_s.at[dst_v.at[j]], add=True)

        for b in range(NB):
            gather(b, b)

        def step(i, carry):
            j = i * NB
            for b in range(NB):
                wait_gather(j + b, b)
                scatter(j + b, b)
                gather(j + b + NB, b)
            return carry

        lax.fori_loop(0, NCH // NB - 1, step, 0)
        j = NCH - NB
        for b in range(NB):
            wait_gather(j + b, b)
            scatter(j + b, b)
        plsc.subcore_barrier()

        pltpu.sync_copy(acc_s.at[pl.ds(base, RPS)], out_p.at[c, pl.ds(base, RPS)])
        if with_deg:
            pltpu.sync_copy(deg_s.at[pl.ds(base, RPS)],
                            out_deg.at[c, pl.ds(base, RPS)])

    return pl.kernel(body, out_type=out_type, scratch_types=scratch, mesh=mesh)


_agg_hid_deg = _make_sc_agg(F_HID, True)
_agg_hid = _make_sc_agg(F_HID, False)


_T_DN = (((0,), (0,)), ((), ()))


def _mm_in_one(xt, w, b=None):
    def body(xt_ref, w_ref, *rest):
        if b is None:
            (o_ref,) = rest
            bias = 0.0
        else:
            b_ref, o_ref = rest
            bias = b_ref[...]
        o_ref[...] = lax.dot_general(
            xt_ref[...], w_ref[...], _T_DN,
            preferred_element_type=jnp.float32) + bias

    in_specs = [
        pl.BlockSpec((F_IN, RB), lambda i: (0, i)),
        pl.BlockSpec((F_IN, F_HID), lambda i: (0, 0)),
    ]
    args = [xt, w]
    if b is not None:
        in_specs.append(pl.BlockSpec((1, F_HID), lambda i: (0, 0)))
        args.append(b.reshape(1, F_HID))
    return pl.pallas_call(
        body,
        grid=(NPAD // RB,),
        in_specs=in_specs,
        out_specs=pl.BlockSpec((RB, F_HID), lambda i: (i, 0)),
        out_shape=jax.ShapeDtypeStruct((NPAD, F_HID), jnp.float32),
    )(*args)


def _combine_mm_one(hs, p, degp, w, fo, b=None):
    def body(hs_ref, p_ref, d_ref, w_ref, *rest):
        if b is None:
            (o_ref,) = rest
            bias = 0.0
        else:
            b_ref, o_ref = rest
            bias = b_ref[...]
        ps = p_ref[0] + p_ref[1]
        d = d_ref[0] + d_ref[1]
        inv = 1.0 / jnp.maximum(d, 1.0)
        h = jnp.maximum(hs_ref[...] + ps * inv[:, None], 0.0)
        o_ref[...] = jnp.dot(h, w_ref[...],
                             preferred_element_type=jnp.float32) + bias

    in_specs = [
        pl.BlockSpec((RB, F_HID), lambda i: (i, 0)),
        pl.BlockSpec((2, RB, F_HID), lambda i: (0, i, 0)),
        pl.BlockSpec((2, RB), lambda i: (0, i)),
        pl.BlockSpec((F_HID, fo), lambda i: (0, 0)),
    ]
    args = [hs, p, degp, w]
    if b is not None:
        in_specs.append(pl.BlockSpec((1, fo), lambda i: (0, 0)))
        args.append(b.reshape(1, fo))
    return pl.pallas_call(
        body,
        grid=(NPAD // RB,),
        in_specs=in_specs,
        out_specs=pl.BlockSpec((RB, fo), lambda i: (i, 0)),
        out_shape=jax.ShapeDtypeStruct((NPAD, fo), jnp.float32),
    )(*args)


def _final_body(hs_ref, p_ref, d_ref, o_ref):
    ps = p_ref[0, :, :F_OUT] + p_ref[1, :, :F_OUT]
    d = d_ref[0] + d_ref[1]
    res = hs_ref[...] + ps * (1.0 / jnp.maximum(d, 1.0))[:, None]
    o_ref[...] = res.T


def _final(hs, p, degp):
    return pl.pallas_call(
        _final_body,
        grid=(NPAD // RB,),
        in_specs=[
            pl.BlockSpec((RB, F_OUT), lambda i: (i, 0)),
            pl.BlockSpec((2, RB, F_HID), lambda i: (0, i, 0)),
            pl.BlockSpec((2, RB), lambda i: (0, i)),
        ],
        out_specs=pl.BlockSpec((F_OUT, RB), lambda i: (0, i)),
        out_shape=jax.ShapeDtypeStruct((F_OUT, N), jnp.float32),
    )(hs, p, degp)


def kernel(x, edge_index, W_self_0, W_neigh_0, b_0, W_self_1, W_neigh_1, b_1,
           W_self_2, W_neigh_2, b_2):
    xt = x.reshape(-1, F_IN).T
    fill = (jnp.arange(EPAD - E, dtype=jnp.int32) % (NPAD - N)) + N
    fill2 = jnp.broadcast_to(fill, (2, EPAD - E))
    edges = jnp.concatenate([edge_index.astype(jnp.int32), fill2],
                            axis=1).reshape(2, NW, NCH, CH)
    wn2 = jnp.pad(W_neigh_2, ((0, 0), (0, F_HID - F_OUT)))

    hn0 = _mm_in_one(xt, W_neigh_0)
    p0, degp = _agg_hid_deg(hn0, edges)
    hs0 = _mm_in_one(xt, W_self_0, b_0)
    hn1 = _combine_mm_one(hs0, p0, degp, W_neigh_1, F_HID)
    (p1,) = _agg_hid(hn1, edges)
    hs1 = _combine_mm_one(hs0, p0, degp, W_self_1, F_HID, b_1)
    hn2 = _combine_mm_one(hs1, p1, degp, wn2, F_HID)
    (p2,) = _agg_hid(hn2, edges)
    hs2 = _combine_mm_one(hs1, p1, degp, W_self_2, F_OUT, b_2)
    out_t = _final(hs2, p2, degp)
    return out_t.T


# speedup vs baseline: 1.3891x; 1.0252x over previous
import jax
import jax.numpy as jnp
from jax import lax
from jax.experimental import pallas as pl
from jax.experimental.pallas import tpu as pltpu
from jax.experimental.pallas import tpu_sc as plsc

N = 10000
NPAD = 10240
E = 160000
EPAD = 163840
NW = 32
EPW = EPAD // NW
CH = 128
NCH = EPW // CH
RPS = NPAD // 16
RB = 2560
F_IN, F_HID, F_OUT = 300, 128, 64


def _make_sc_agg(F, with_deg):
    mesh = plsc.VectorSubcoreMesh(core_axis_name="c", subcore_axis_name="s")
    out_type = [jax.ShapeDtypeStruct((2, NPAD, F), jnp.float32)]
    scratch = [
        pltpu.VMEM_SHARED((NPAD, F), jnp.float32),
        pltpu.VMEM((NCH, CH), jnp.int32),
        pltpu.VMEM((NCH, CH), jnp.int32),
        pltpu.VMEM((CH, F), jnp.float32),
        pltpu.VMEM((CH, F), jnp.float32),
        pltpu.SemaphoreType.DMA,
        pltpu.SemaphoreType.DMA,
    ]
    if with_deg:
        out_type.append(jax.ShapeDtypeStruct((2, NPAD), jnp.float32))
        scratch += [
            pltpu.VMEM_SHARED((NPAD,), jnp.float32),
            pltpu.VMEM((CH,), jnp.float32),
        ]

    def body(*refs):
        if with_deg:
            (hn, edges, out_p, out_deg,
             acc_s, src_v, dst_v, rows0, rows1,
             semg0, semg1, deg_s, ones_v) = refs
        else:
            (hn, edges, out_p,
             acc_s, src_v, dst_v, rows0, rows1,
             semg0, semg1) = refs
        bufs = (rows0, rows1)
        gsems = (semg0, semg1)
        NB = 2
        c = lax.axis_index("c")
        s = lax.axis_index("s")
        wid = s * 2 + c
        base = s * RPS

        def gather(j, b):
            pltpu.async_copy(hn.at[src_v.at[j]], bufs[b], gsems[b])

        def wait_gather(j, b):
            pltpu.make_async_copy(hn.at[src_v.at[j]], bufs[b], gsems[b]).wait()

        def scatter(j, b):
            pltpu.sync_copy(bufs[b], acc_s.at[dst_v.at[j]], add=True)
            if with_deg:
                pltpu.sync_copy(ones_v, deg_s.at[dst_v.at[j]], add=True)

        pltpu.sync_copy(edges.at[0, wid], src_v)
        pltpu.sync_copy(edges.at[1, wid], dst_v)
        gather(0, 0)

        def zrow(j, carry):
            for k in range(F // 16):
                rows1[j, pl.ds(k * 16, 16)] = jnp.zeros((16,), jnp.float32)
            return carry
        lax.fori_loop(0, CH, zrow, 0)
        for m in range(RPS // CH):
            pltpu.sync_copy(rows1, acc_s.at[pl.ds(base + m * CH, CH)])
        if with_deg:
            for m in range(RPS // CH):
                pltpu.sync_copy(rows1.at[0], deg_s.at[pl.ds(base + m * CH, CH)])
            for i in range(CH // 16):
                ones_v[pl.ds(i * 16, 16)] = jnp.ones((16,), jnp.float32)
        plsc.subcore_barrier()
        gather(1, 1)

        def step(i, carry):
            j = i * NB
            for b in range(NB):
                wait_gather(j + b, b)
                scatter(j + b, b)
                gather(j + b + NB, b)
            return carry

        lax.fori_loop(0, NCH // NB - 1, step, 0)
        j = NCH - NB
        for b in range(NB):
            wait_gather(j + b, b)
            scatter(j + b, b)
        plsc.subcore_barrier()

        pltpu.sync_copy(acc_s.at[pl.ds(base, RPS)], out_p.at[c, pl.ds(base, RPS)])
        if with_deg:
            pltpu.sync_copy(deg_s.at[pl.ds(base, RPS)],
                            out_deg.at[c, pl.ds(base, RPS)])

    return pl.kernel(body, out_type=out_type, scratch_types=scratch, mesh=mesh)


_agg_hid_deg = _make_sc_agg(F_HID, True)
_agg_hid = _make_sc_agg(F_HID, False)


_T_DN = (((0,), (0,)), ((), ()))


def _mm_in_one(xt, w, b=None):
    def body(xt_ref, w_ref, *rest):
        if b is None:
            (o_ref,) = rest
            bias = 0.0
        else:
            b_ref, o_ref = rest
            bias = b_ref[...]
        o_ref[...] = lax.dot_general(
            xt_ref[...], w_ref[...], _T_DN,
            preferred_element_type=jnp.float32) + bias

    in_specs = [
        pl.BlockSpec((F_IN, RB), lambda i: (0, i)),
        pl.BlockSpec((F_IN, F_HID), lambda i: (0, 0)),
    ]
    args = [xt, w]
    if b is not None:
        in_specs.append(pl.BlockSpec((1, F_HID), lambda i: (0, 0)))
        args.append(b.reshape(1, F_HID))
    return pl.pallas_call(
        body,
        grid=(NPAD // RB,),
        in_specs=in_specs,
        out_specs=pl.BlockSpec((RB, F_HID), lambda i: (i, 0)),
        out_shape=jax.ShapeDtypeStruct((NPAD, F_HID), jnp.float32),
    )(*args)


def _combine_mm_one(hs, p, degp, w, fo, b=None):
    def body(hs_ref, p_ref, d_ref, w_ref, *rest):
        if b is None:
            (o_ref,) = rest
            bias = 0.0
        else:
            b_ref, o_ref = rest
            bias = b_ref[...]
        ps = p_ref[0] + p_ref[1]
        d = d_ref[0] + d_ref[1]
        inv = 1.0 / jnp.maximum(d, 1.0)
        h = jnp.maximum(hs_ref[...] + ps * inv[:, None], 0.0)
        o_ref[...] = jnp.dot(h, w_ref[...],
                             preferred_element_type=jnp.float32) + bias

    in_specs = [
        pl.BlockSpec((RB, F_HID), lambda i: (i, 0)),
        pl.BlockSpec((2, RB, F_HID), lambda i: (0, i, 0)),
        pl.BlockSpec((2, RB), lambda i: (0, i)),
        pl.BlockSpec((F_HID, fo), lambda i: (0, 0)),
    ]
    args = [hs, p, degp, w]
    if b is not None:
        in_specs.append(pl.BlockSpec((1, fo), lambda i: (0, 0)))
        args.append(b.reshape(1, fo))
    return pl.pallas_call(
        body,
        grid=(NPAD // RB,),
        in_specs=in_specs,
        out_specs=pl.BlockSpec((RB, fo), lambda i: (i, 0)),
        out_shape=jax.ShapeDtypeStruct((NPAD, fo), jnp.float32),
    )(*args)


def _final_body(hs_ref, p_ref, d_ref, o_ref):
    ps = p_ref[0, :, :F_OUT] + p_ref[1, :, :F_OUT]
    d = d_ref[0] + d_ref[1]
    res = hs_ref[...] + ps * (1.0 / jnp.maximum(d, 1.0))[:, None]
    o_ref[...] = res.T


def _final(hs, p, degp):
    return pl.pallas_call(
        _final_body,
        grid=(NPAD // RB,),
        in_specs=[
            pl.BlockSpec((RB, F_OUT), lambda i: (i, 0)),
            pl.BlockSpec((2, RB, F_HID), lambda i: (0, i, 0)),
            pl.BlockSpec((2, RB), lambda i: (0, i)),
        ],
        out_specs=pl.BlockSpec((F_OUT, RB), lambda i: (0, i)),
        out_shape=jax.ShapeDtypeStruct((F_OUT, N), jnp.float32),
    )(hs, p, degp)


def kernel(x, edge_index, W_self_0, W_neigh_0, b_0, W_self_1, W_neigh_1, b_1,
           W_self_2, W_neigh_2, b_2):
    xt = x.reshape(-1, F_IN).T
    fill = (jnp.arange(EPAD - E, dtype=jnp.int32) % (NPAD - N)) + N
    fill2 = jnp.broadcast_to(fill, (2, EPAD - E))
    edges = jnp.concatenate([edge_index.astype(jnp.int32), fill2],
                            axis=1).reshape(2, NW, NCH, CH)
    wn2 = jnp.pad(W_neigh_2, ((0, 0), (0, F_HID - F_OUT)))

    hn0 = _mm_in_one(xt, W_neigh_0)
    p0, degp = _agg_hid_deg(hn0, edges)
    hs0 = _mm_in_one(xt, W_self_0, b_0)
    hn1 = _combine_mm_one(hs0, p0, degp, W_neigh_1, F_HID)
    (p1,) = _agg_hid(hn1, edges)
    hs1 = _combine_mm_one(hs0, p0, degp, W_self_1, F_HID, b_1)
    hn2 = _combine_mm_one(hs1, p1, degp, wn2, F_HID)
    (p2,) = _agg_hid(hn2, edges)
    hs2 = _combine_mm_one(hs1, p1, degp, W_self_2, F_OUT, b_2)
    out_t = _final(hs2, p2, degp)
    return out_t.T
